# baseline (speedup 1.0000x reference)
import functools

import jax
import jax.numpy as jnp
from jax.experimental import pallas as pl
from jax.experimental.pallas import tpu as pltpu

_S = 3.0
_M = 0.2


def _loss_kernel(x_ref, t_ref, out_ref, *, R, C, B):
    i = pl.program_id(0)

    @pl.when(i == 0)
    def _init():
        out_ref[...] = jnp.zeros((1, 1), jnp.float32)

    x = x_ref[...]
    m = jnp.max(x, axis=1, keepdims=True)
    s = jnp.sum(jnp.exp(_S * x - _S * m), axis=1, keepdims=True)

    cols = jax.lax.broadcasted_iota(jnp.int32, (R, C), 1)
    t = t_ref[...]
    tv = jnp.sum(jnp.where(cols == t, x, 0.0), axis=1, keepdims=True)

    out_t = _S * tv - _S * _M
    s_c = s - jnp.exp(_S * tv - _S * m) + jnp.exp(out_t - _S * m)
    loss = _S * m + jnp.log(s_c) - out_t
    out_ref[...] += (jnp.sum(loss) / B).reshape(1, 1)


def kernel(cos_theta, cos_theta_aux, target):
    B, C = cos_theta.shape
    R = 32
    out = pl.pallas_call(
        functools.partial(_loss_kernel, R=R, C=C, B=B),
        grid=(B // R,),
        in_specs=[
            pl.BlockSpec((R, C), lambda i: (i, 0)),
            pl.BlockSpec((R, 1), lambda i: (i, 0)),
        ],
        out_specs=pl.BlockSpec((1, 1), lambda i: (0, 0)),
        out_shape=jax.ShapeDtypeStruct((1, 1), jnp.float32),
    )(cos_theta, target.reshape(B, 1).astype(jnp.int32))
    return out[0, 0]
